# baseline (speedup 1.0000x reference)
import functools

import jax
import jax.numpy as jnp
from jax import lax
from jax.experimental import pallas as pl
from jax.experimental.pallas import tpu as pltpu
from jax.experimental.pallas import tpu_sc as plsc

N = 10000
E = 160000
IN_DIM = 128
EDGE_DIM = 4
HID = 8
G = 512
NUM_LAYERS = 3
EPS = 1e-5

NW = 32
C = 128
KE = 40
EPAD = NW * KE * C
CC = 80
KC = 64
NGRP = KC // 8
KN = 3
NPAD = NW * KN * C
SN = 10112
SG = 640
DEAD_N = N
DEAD_G = G

HIGH = jax.lax.Precision.HIGHEST


def _sc_gather(table, idx3, D):
    kc = idx3.shape[1]
    ngrp = kc // 8
    M = NW * kc * CC
    dt = table.dtype
    mesh = plsc.VectorSubcoreMesh(core_axis_name="c", subcore_axis_name="s")

    @functools.partial(
        pl.kernel,
        mesh=mesh,
        out_type=jax.ShapeDtypeStruct((M, D), dt),
        compiler_params=pltpu.CompilerParams(use_tc_tiling_on_sc=False),
        scratch_types=(
            [pltpu.VMEM((kc, CC), jnp.int32)]
            + [pltpu.VMEM((CC, D), dt)] * 16
            + [pltpu.SemaphoreType.DMA, pltpu.SemaphoreType.DMA]
        ),
    )
    def k(table_hbm, idx_hbm, out_hbm, idx_v, *rest):
        bufs = rest[:16]
        gsem, wsem = rest[16], rest[17]
        cid = lax.axis_index("c")
        sid = lax.axis_index("s")
        wid = cid * 16 + sid
        pltpu.sync_copy(idx_hbm.at[wid], idx_v)
        base = wid * (kc * CC)
        half = (bufs[:8], bufs[8:])
        for b in range(8):
            pltpu.async_copy(table_hbm.at[idx_v.at[b]], bufs[b], gsem)

        def body(g, _):
            for par in range(2):
                @pl.when(lax.rem(g, 2) == par)
                def _():
                    cur = half[par]
                    nxt = half[1 - par]
                    for b in range(8):
                        pltpu.make_async_copy(
                            table_hbm.at[idx_v.at[0]], cur[b], gsem).wait()
                    @pl.when(g + 1 < ngrp)
                    def _():
                        @pl.when(g >= 1)
                        def _():
                            for b in range(8):
                                pltpu.make_async_copy(
                                    nxt[b], out_hbm.at[pl.ds(base, CC)],
                                    wsem).wait()
                        for b in range(8):
                            pltpu.async_copy(
                                table_hbm.at[idx_v.at[(g + 1) * 8 + b]],
                                nxt[b], gsem)
                    for b in range(8):
                        pltpu.async_copy(
                            cur[b],
                            out_hbm.at[pl.ds(base + (g * 8 + b) * CC, CC)],
                            wsem)
            return 0

        lax.fori_loop(0, ngrp, body, 0)
        for b in range(16):
            pltpu.make_async_copy(
                bufs[0], out_hbm.at[pl.ds(base, CC)], wsem).wait()

    return k(table, idx3)


def _sc_scatter_add16(values, idx3, zeros, K, S):
    PW = K * C
    TR = S // 16
    mesh = plsc.VectorSubcoreMesh(core_axis_name="c", subcore_axis_name="s")

    @functools.partial(
        pl.kernel,
        mesh=mesh,
        out_type=jax.ShapeDtypeStruct((2, S, 16), jnp.float32),
        compiler_params=pltpu.CompilerParams(use_tc_tiling_on_sc=False),
        scratch_types=[
            pltpu.VMEM((K, C), jnp.int32),
            pltpu.VMEM((PW, 16), jnp.float32),
            pltpu.VMEM((TR, 16), jnp.float32),
            pltpu.VMEM_SHARED((S, 16), jnp.float32),
        ],
    )
    def k(val_hbm, idx_hbm, z_hbm, out_hbm, idx_v, val_v, row_v, acc_sh):
        cid = lax.axis_index("c")
        sid = lax.axis_index("s")
        wid = cid * 16 + sid
        pltpu.sync_copy(z_hbm.at[pl.ds(sid * TR, TR)], row_v)
        pltpu.sync_copy(row_v, acc_sh.at[pl.ds(sid * TR, TR)])
        pltpu.sync_copy(idx_hbm.at[wid], idx_v)
        pltpu.sync_copy(val_hbm.at[pl.ds(wid * PW, PW)], val_v)
        plsc.subcore_barrier()

        def body(j, _):
            pltpu.sync_copy(val_v.at[pl.ds(j * C, C)],
                            acc_sh.at[idx_v.at[j]], add=True)
            return 0

        lax.fori_loop(0, K, body, 0)
        plsc.subcore_barrier()
        pltpu.sync_copy(acc_sh.at[pl.ds(sid * TR, TR)], row_v)
        pltpu.sync_copy(row_v, out_hbm.at[cid, pl.ds(sid * TR, TR)])

    return k(values, idx3, zeros)


def _msg_body(hs_ref, ea_ref, wp_ref, out_ref, *, din, dp):
    B = out_ref.shape[0]
    lanes = 8 * dp
    ea = ea_ref[...]
    w = jnp.dot(ea, wp_ref[...])
    w = jnp.maximum(w, 0.0).astype(jnp.bfloat16)
    hs = hs_ref[...].astype(jnp.bfloat16)
    if din < dp:
        hs = jnp.concatenate(
            [hs, jnp.zeros((B, dp - din), jnp.bfloat16)], axis=1)
    if dp == 128:
        t = jnp.concatenate(
            [hs * w[:, o * dp:(o + 1) * dp] for o in range(8)], axis=1)
    else:
        t = jnp.concatenate([hs] * 8, axis=1) * w
    ci = jax.lax.broadcasted_iota(jnp.int32, (lanes, 16), 0)
    oi = jax.lax.broadcasted_iota(jnp.int32, (lanes, 16), 1)
    sel = jnp.where((ci // dp) == oi, 1.0, 0.0).astype(jnp.bfloat16)
    msg = jnp.dot(t, sel, preferred_element_type=jnp.float32)
    onescol = (jax.lax.broadcasted_iota(jnp.int32, (B, 16), 1) == 8)
    out_ref[...] = msg + onescol.astype(jnp.float32)


def _tc_messages(hsrc, ea8, wp, din, dp):
    M = hsrc.shape[0]
    B = 2048 if dp == 128 else 8192
    grid = M // B
    return pl.pallas_call(
        functools.partial(_msg_body, din=din, dp=dp),
        grid=(grid,),
        in_specs=[
            pl.BlockSpec((B, din), lambda i: (i, 0)),
            pl.BlockSpec((B, 8), lambda i: (i, 0)),
            pl.BlockSpec((8, 8 * dp), lambda i: (0, 0)),
        ],
        out_specs=pl.BlockSpec((B, 16), lambda i: (i, 0)),
        out_shape=jax.ShapeDtypeStruct((M, 16), jnp.float32),
    )(hsrc, ea8, wp)


def _combine_body(pa_ref, pb_ref, h_ref, root_ref, bias_ref, gamma_ref,
                  beta_ref, out_ref, *, din):
    s = pa_ref[0] + pa_ref[1] + pb_ref[0] + pb_ref[1]
    aggr = s[0:N, 0:8]
    deg = s[0:N, 8:9]
    denom = jnp.maximum(deg, 1.0)
    hp = h_ref[0:N, 0:din]
    out = aggr / denom + jnp.dot(hp, root_ref[...], precision=HIGH)
    out = out + bias_ref[...]
    mu = jnp.mean(out, axis=0, keepdims=True)
    var = jnp.mean((out - mu) ** 2, axis=0, keepdims=True)
    outn = (out - mu) * jax.lax.rsqrt(var + EPS)
    outn = outn * gamma_ref[...] + beta_ref[...]
    h = jnp.maximum(outn, 0.0)
    onescol = (jax.lax.broadcasted_iota(jnp.int32, (N, 8), 1) == 0)
    h16 = jnp.concatenate([h, onescol.astype(jnp.float32)], axis=1)
    pad = jnp.zeros((NPAD - N, 16), jnp.float32)
    out_ref[...] = jnp.concatenate([h16, pad], axis=0)


def _tc_combine(pa, pb, h_prev, rootp, bias, gamma, beta, din, hrows):
    return pl.pallas_call(
        functools.partial(_combine_body, din=din),
        compiler_params=pltpu.CompilerParams(
            vmem_limit_bytes=100 * 1024 * 1024),
        in_specs=[
            pl.BlockSpec((2, SN, 16), lambda: (0, 0, 0)),
            pl.BlockSpec((2, SN, 16), lambda: (0, 0, 0)),
            pl.BlockSpec((hrows, din), lambda: (0, 0)),
            pl.BlockSpec((din, HID), lambda: (0, 0)),
            pl.BlockSpec((1, HID), lambda: (0, 0)),
            pl.BlockSpec((1, HID), lambda: (0, 0)),
            pl.BlockSpec((1, HID), lambda: (0, 0)),
        ],
        out_specs=pl.BlockSpec((NPAD, 16), lambda: (0, 0)),
        out_shape=jax.ShapeDtypeStruct((NPAD, 16), jnp.float32),
    )(pa, pb, h_prev, rootp, bias, gamma, beta)


def _head_body(pp_ref, w_ref, b_ref, out_ref):
    s = pp_ref[0] + pp_ref[1]
    pooled = s[0:G, 0:8] / jnp.maximum(s[0:G, 8:9], 1.0)
    out_ref[...] = jnp.dot(pooled, w_ref[...], precision=HIGH) + b_ref[...]


def _tc_head(pp, headw, headb, n_tasks):
    return pl.pallas_call(
        _head_body,
        in_specs=[
            pl.BlockSpec((2, SG, 16), lambda: (0, 0, 0)),
            pl.BlockSpec((HID, n_tasks), lambda: (0, 0)),
            pl.BlockSpec((1, n_tasks), lambda: (0, 0)),
        ],
        out_specs=pl.BlockSpec((G, n_tasks), lambda: (0, 0)),
        out_shape=jax.ShapeDtypeStruct((G, n_tasks), jnp.float32),
    )(pp, headw, headb)


def _perm_edge_weights(we, be, in_ch, dp):
    we3 = we.reshape(EDGE_DIM, in_ch, HID)
    wp = jnp.transpose(we3, (0, 2, 1))
    wp = jnp.pad(wp, ((0, 0), (0, 0), (0, dp - in_ch)))
    bp = jnp.pad(be.reshape(in_ch, HID).T, ((0, 0), (0, dp - in_ch)))
    return jnp.concatenate([
        wp.reshape(EDGE_DIM, 8 * dp),
        bp.reshape(1, 8 * dp),
        jnp.zeros((3, 8 * dp), jnp.float32),
    ], axis=0)


def kernel(x, edge_index, edge_attr, batch, params):
    src = edge_index[0]
    dst = edge_index[1]
    EH = EPAD // 2
    src_f = jnp.pad(src, (0, EPAD - E))
    dst_f = jnp.pad(dst, (0, EPAD - E), constant_values=DEAD_N)
    src_h = [src_f[p * EH:(p + 1) * EH].reshape(NW, EH // NW // CC, CC)
             for p in range(2)]
    dst_h = [dst_f[p * EH:(p + 1) * EH].reshape(NW, EH // NW // C, C)
             for p in range(2)]
    ea8 = jnp.concatenate([
        edge_attr,
        jnp.ones((E, 1), jnp.float32),
        jnp.zeros((E, 3), jnp.float32),
    ], axis=1)
    ea8 = jnp.pad(ea8, ((0, EPAD - E), (0, 0)))
    ea8_h = [ea8[p * EH:(p + 1) * EH] for p in range(2)]
    batch_p = jnp.pad(batch, (0, NPAD - N),
                      constant_values=DEAD_G).reshape(NW, KN, C)
    z_n = jnp.zeros((SN, 16), jnp.float32)
    z_g = jnp.zeros((SG, 16), jnp.float32)

    h = x
    din = IN_DIM
    hrows = N
    for i in range(NUM_LAYERS):
        dp = 128 if din == IN_DIM else 16
        dgather = din if din == IN_DIM else 16
        wp = _perm_edge_weights(params["We"][i], params["be"][i], din, dp)
        rootp = params["root"][i]
        if din != IN_DIM:
            rootp = jnp.pad(rootp, ((0, 16 - HID), (0, 0)))
        htab = h.astype(jnp.bfloat16) if din == IN_DIM else h
        hsrc_a = _sc_gather(htab, src_h[0], dgather)
        hsrc_b = _sc_gather(htab, src_h[1], dgather)
        msg_a = _tc_messages(hsrc_a, ea8_h[0], wp, dgather, dp)
        msg_b = _tc_messages(hsrc_b, ea8_h[1], wp, dgather, dp)
        part_a = _sc_scatter_add16(msg_a, dst_h[0], z_n, EH // NW // C, SN)
        part_b = _sc_scatter_add16(msg_b, dst_h[1], z_n, EH // NW // C, SN)
        h = _tc_combine(
            part_a, part_b, h, rootp,
            params["bias"][i].reshape(1, HID),
            params["gamma"][i].reshape(1, HID),
            params["beta"][i].reshape(1, HID),
            din if din == IN_DIM else 16, hrows)
        din = HID
        hrows = NPAD
    pp = _sc_scatter_add16(h, batch_p, z_g, KN, SG)
    n_tasks = params["headW"].shape[1]
    return _tc_head(pp, params["headW"], params["headb"].reshape(1, n_tasks),
                    n_tasks)


# speedup vs baseline: 1.0670x; 1.0670x over previous
import functools

import jax
import jax.numpy as jnp
from jax import lax
from jax.experimental import pallas as pl
from jax.experimental.pallas import tpu as pltpu
from jax.experimental.pallas import tpu_sc as plsc

N = 10000
E = 160000
IN_DIM = 128
EDGE_DIM = 4
HID = 8
G = 512
NUM_LAYERS = 3
EPS = 1e-5

NW = 32
C = 128
KE = 40
EPAD = NW * KE * C
CC = 40
KC = 128
NGRP = KC // 8
KN = 3
NPAD = NW * KN * C
SN = 10112
SG = 640
DEAD_N = N
DEAD_G = G

HIGH = jax.lax.Precision.HIGHEST


def _sc_gather(table, idx3, D):
    kc = idx3.shape[1]
    ngrp = kc // 8
    M = NW * kc * CC
    dt = table.dtype
    mesh = plsc.VectorSubcoreMesh(core_axis_name="c", subcore_axis_name="s")

    @functools.partial(
        pl.kernel,
        mesh=mesh,
        out_type=jax.ShapeDtypeStruct((M, D), dt),
        compiler_params=pltpu.CompilerParams(use_tc_tiling_on_sc=False),
        scratch_types=(
            [pltpu.VMEM((kc, CC), jnp.int32)]
            + [pltpu.VMEM((CC, D), dt)] * 16
            + [pltpu.SemaphoreType.DMA, pltpu.SemaphoreType.DMA]
        ),
    )
    def k(table_hbm, idx_hbm, out_hbm, idx_v, *rest):
        bufs = rest[:16]
        gsem, wsem = rest[16], rest[17]
        cid = lax.axis_index("c")
        sid = lax.axis_index("s")
        wid = cid * 16 + sid
        pltpu.sync_copy(idx_hbm.at[wid], idx_v)
        base = wid * (kc * CC)
        half = (bufs[:8], bufs[8:])
        for b in range(8):
            pltpu.async_copy(table_hbm.at[idx_v.at[b]], bufs[b], gsem)

        def body(g, _):
            for par in range(2):
                @pl.when(lax.rem(g, 2) == par)
                def _():
                    cur = half[par]
                    nxt = half[1 - par]
                    for b in range(8):
                        pltpu.make_async_copy(
                            table_hbm.at[idx_v.at[0]], cur[b], gsem).wait()
                    @pl.when(g + 1 < ngrp)
                    def _():
                        @pl.when(g >= 1)
                        def _():
                            for b in range(8):
                                pltpu.make_async_copy(
                                    nxt[b], out_hbm.at[pl.ds(base, CC)],
                                    wsem).wait()
                        for b in range(8):
                            pltpu.async_copy(
                                table_hbm.at[idx_v.at[(g + 1) * 8 + b]],
                                nxt[b], gsem)
                    for b in range(8):
                        pltpu.async_copy(
                            cur[b],
                            out_hbm.at[pl.ds(base + (g * 8 + b) * CC, CC)],
                            wsem)
            return 0

        lax.fori_loop(0, ngrp, body, 0)
        for b in range(16):
            pltpu.make_async_copy(
                bufs[0], out_hbm.at[pl.ds(base, CC)], wsem).wait()

    return k(table, idx3)


def _sc_scatter_add16(values, idx3, zeros, K, S):
    PW = K * C
    TR = S // 16
    mesh = plsc.VectorSubcoreMesh(core_axis_name="c", subcore_axis_name="s")

    @functools.partial(
        pl.kernel,
        mesh=mesh,
        out_type=jax.ShapeDtypeStruct((2, S, 16), jnp.float32),
        compiler_params=pltpu.CompilerParams(use_tc_tiling_on_sc=False),
        scratch_types=[
            pltpu.VMEM((K, C), jnp.int32),
            pltpu.VMEM((PW, 16), jnp.float32),
            pltpu.VMEM((TR, 16), jnp.float32),
            pltpu.VMEM_SHARED((S, 16), jnp.float32),
        ],
    )
    def k(val_hbm, idx_hbm, z_hbm, out_hbm, idx_v, val_v, row_v, acc_sh):
        cid = lax.axis_index("c")
        sid = lax.axis_index("s")
        wid = cid * 16 + sid
        pltpu.sync_copy(z_hbm.at[pl.ds(sid * TR, TR)], row_v)
        pltpu.sync_copy(row_v, acc_sh.at[pl.ds(sid * TR, TR)])
        pltpu.sync_copy(idx_hbm.at[wid], idx_v)
        pltpu.sync_copy(val_hbm.at[pl.ds(wid * PW, PW)], val_v)
        plsc.subcore_barrier()

        def body(j, _):
            pltpu.sync_copy(val_v.at[pl.ds(j * C, C)],
                            acc_sh.at[idx_v.at[j]], add=True)
            return 0

        lax.fori_loop(0, K, body, 0)
        plsc.subcore_barrier()
        pltpu.sync_copy(acc_sh.at[pl.ds(sid * TR, TR)], row_v)
        pltpu.sync_copy(row_v, out_hbm.at[cid, pl.ds(sid * TR, TR)])

    return k(values, idx3, zeros)


def _msg_body(hs_ref, ea_ref, wp_ref, out_ref, *, din, dp):
    B = out_ref.shape[0]
    lanes = 8 * dp
    ea = ea_ref[...]
    w = jnp.dot(ea, wp_ref[...])
    w = jnp.maximum(w, 0.0).astype(jnp.bfloat16)
    hs = hs_ref[...].astype(jnp.bfloat16)
    if din < dp:
        hs = jnp.concatenate(
            [hs, jnp.zeros((B, dp - din), jnp.bfloat16)], axis=1)
    if dp == 128:
        t = jnp.concatenate(
            [hs * w[:, o * dp:(o + 1) * dp] for o in range(8)], axis=1)
    else:
        t = jnp.concatenate([hs] * 8, axis=1) * w
    ci = jax.lax.broadcasted_iota(jnp.int32, (lanes, 16), 0)
    oi = jax.lax.broadcasted_iota(jnp.int32, (lanes, 16), 1)
    sel = jnp.where((ci // dp) == oi, 1.0, 0.0).astype(jnp.bfloat16)
    msg = jnp.dot(t, sel, preferred_element_type=jnp.float32)
    onescol = (jax.lax.broadcasted_iota(jnp.int32, (B, 16), 1) == 8)
    out_ref[...] = msg + onescol.astype(jnp.float32)


def _tc_messages(hsrc, ea8, wp, din, dp):
    M = hsrc.shape[0]
    B = 2048 if dp == 128 else 8192
    grid = M // B
    return pl.pallas_call(
        functools.partial(_msg_body, din=din, dp=dp),
        grid=(grid,),
        in_specs=[
            pl.BlockSpec((B, din), lambda i: (i, 0)),
            pl.BlockSpec((B, 8), lambda i: (i, 0)),
            pl.BlockSpec((8, 8 * dp), lambda i: (0, 0)),
        ],
        out_specs=pl.BlockSpec((B, 16), lambda i: (i, 0)),
        out_shape=jax.ShapeDtypeStruct((M, 16), jnp.float32),
    )(hsrc, ea8, wp)


def _combine_body(pa_ref, pb_ref, h_ref, root_ref, bias_ref, gamma_ref,
                  beta_ref, out_ref, *, din):
    s = pa_ref[0] + pa_ref[1] + pb_ref[0] + pb_ref[1]
    aggr = s[0:N, 0:8]
    deg = s[0:N, 8:9]
    denom = jnp.maximum(deg, 1.0)
    hp = h_ref[0:N, 0:din]
    out = aggr / denom + jnp.dot(hp, root_ref[...], precision=HIGH)
    out = out + bias_ref[...]
    mu = jnp.mean(out, axis=0, keepdims=True)
    var = jnp.mean((out - mu) ** 2, axis=0, keepdims=True)
    outn = (out - mu) * jax.lax.rsqrt(var + EPS)
    outn = outn * gamma_ref[...] + beta_ref[...]
    h = jnp.maximum(outn, 0.0)
    onescol = (jax.lax.broadcasted_iota(jnp.int32, (N, 8), 1) == 0)
    h16 = jnp.concatenate([h, onescol.astype(jnp.float32)], axis=1)
    pad = jnp.zeros((NPAD - N, 16), jnp.float32)
    out_ref[...] = jnp.concatenate([h16, pad], axis=0)


def _tc_combine(pa, pb, h_prev, rootp, bias, gamma, beta, din, hrows):
    return pl.pallas_call(
        functools.partial(_combine_body, din=din),
        compiler_params=pltpu.CompilerParams(
            vmem_limit_bytes=100 * 1024 * 1024),
        in_specs=[
            pl.BlockSpec((2, SN, 16), lambda: (0, 0, 0)),
            pl.BlockSpec((2, SN, 16), lambda: (0, 0, 0)),
            pl.BlockSpec((hrows, din), lambda: (0, 0)),
            pl.BlockSpec((din, HID), lambda: (0, 0)),
            pl.BlockSpec((1, HID), lambda: (0, 0)),
            pl.BlockSpec((1, HID), lambda: (0, 0)),
            pl.BlockSpec((1, HID), lambda: (0, 0)),
        ],
        out_specs=pl.BlockSpec((NPAD, 16), lambda: (0, 0)),
        out_shape=jax.ShapeDtypeStruct((NPAD, 16), jnp.float32),
    )(pa, pb, h_prev, rootp, bias, gamma, beta)


def _head_body(pp_ref, w_ref, b_ref, out_ref):
    s = pp_ref[0] + pp_ref[1]
    pooled = s[0:G, 0:8] / jnp.maximum(s[0:G, 8:9], 1.0)
    out_ref[...] = jnp.dot(pooled, w_ref[...], precision=HIGH) + b_ref[...]


def _tc_head(pp, headw, headb, n_tasks):
    return pl.pallas_call(
        _head_body,
        in_specs=[
            pl.BlockSpec((2, SG, 16), lambda: (0, 0, 0)),
            pl.BlockSpec((HID, n_tasks), lambda: (0, 0)),
            pl.BlockSpec((1, n_tasks), lambda: (0, 0)),
        ],
        out_specs=pl.BlockSpec((G, n_tasks), lambda: (0, 0)),
        out_shape=jax.ShapeDtypeStruct((G, n_tasks), jnp.float32),
    )(pp, headw, headb)


def _perm_edge_weights(we, be, in_ch, dp):
    we3 = we.reshape(EDGE_DIM, in_ch, HID)
    wp = jnp.transpose(we3, (0, 2, 1))
    wp = jnp.pad(wp, ((0, 0), (0, 0), (0, dp - in_ch)))
    bp = jnp.pad(be.reshape(in_ch, HID).T, ((0, 0), (0, dp - in_ch)))
    return jnp.concatenate([
        wp.reshape(EDGE_DIM, 8 * dp),
        bp.reshape(1, 8 * dp),
        jnp.zeros((3, 8 * dp), jnp.float32),
    ], axis=0)


def kernel(x, edge_index, edge_attr, batch, params):
    src = edge_index[0]
    dst = edge_index[1]
    EH = EPAD // 2
    src_f = jnp.pad(src, (0, EPAD - E))
    dst_f = jnp.pad(dst, (0, EPAD - E), constant_values=DEAD_N)
    src_h = [src_f[p * EH:(p + 1) * EH].reshape(NW, EH // NW // CC, CC)
             for p in range(2)]
    dst_h = [dst_f[p * EH:(p + 1) * EH].reshape(NW, EH // NW // C, C)
             for p in range(2)]
    ea8 = jnp.concatenate([
        edge_attr,
        jnp.ones((E, 1), jnp.float32),
        jnp.zeros((E, 3), jnp.float32),
    ], axis=1)
    ea8 = jnp.pad(ea8, ((0, EPAD - E), (0, 0)))
    ea8_h = [ea8[p * EH:(p + 1) * EH] for p in range(2)]
    batch_p = jnp.pad(batch, (0, NPAD - N),
                      constant_values=DEAD_G).reshape(NW, KN, C)
    z_n = jnp.zeros((SN, 16), jnp.float32)
    z_g = jnp.zeros((SG, 16), jnp.float32)

    h = x
    din = IN_DIM
    hrows = N
    for i in range(NUM_LAYERS):
        dp = 128 if din == IN_DIM else 16
        dgather = din if din == IN_DIM else 16
        wp = _perm_edge_weights(params["We"][i], params["be"][i], din, dp)
        rootp = params["root"][i]
        if din != IN_DIM:
            rootp = jnp.pad(rootp, ((0, 16 - HID), (0, 0)))
        hsrc_a = _sc_gather(h, src_h[0], dgather)
        hsrc_b = _sc_gather(h, src_h[1], dgather)
        msg_a = _tc_messages(hsrc_a, ea8_h[0], wp, dgather, dp)
        msg_b = _tc_messages(hsrc_b, ea8_h[1], wp, dgather, dp)
        part_a = _sc_scatter_add16(msg_a, dst_h[0], z_n, EH // NW // C, SN)
        part_b = _sc_scatter_add16(msg_b, dst_h[1], z_n, EH // NW // C, SN)
        h = _tc_combine(
            part_a, part_b, h, rootp,
            params["bias"][i].reshape(1, HID),
            params["gamma"][i].reshape(1, HID),
            params["beta"][i].reshape(1, HID),
            din if din == IN_DIM else 16, hrows)
        din = HID
        hrows = NPAD
    pp = _sc_scatter_add16(h, batch_p, z_g, KN, SG)
    n_tasks = params["headW"].shape[1]
    return _tc_head(pp, params["headW"], params["headb"].reshape(1, n_tasks),
                    n_tasks)


# speedup vs baseline: 1.0781x; 1.0104x over previous
import functools

import jax
import jax.numpy as jnp
from jax import lax
from jax.experimental import pallas as pl
from jax.experimental.pallas import tpu as pltpu
from jax.experimental.pallas import tpu_sc as plsc

N = 10000
E = 160000
IN_DIM = 128
EDGE_DIM = 4
HID = 8
G = 512
NUM_LAYERS = 3
EPS = 1e-5

NW = 32
C = 128
KE = 40
EPAD = NW * KE * C
CC = 40
KC = 128
NGRP = KC // 8
KN = 3
NPAD = NW * KN * C
SN = 10112
SG = 640
DEAD_N = N
DEAD_G = G

HIGH = jax.lax.Precision.HIGHEST


def _sc_gather(table, idx3, D):
    kc = idx3.shape[1]
    ngrp = kc // 8
    M = NW * kc * CC
    dt = table.dtype
    mesh = plsc.VectorSubcoreMesh(core_axis_name="c", subcore_axis_name="s")

    @functools.partial(
        pl.kernel,
        mesh=mesh,
        out_type=jax.ShapeDtypeStruct((M, D), dt),
        compiler_params=pltpu.CompilerParams(use_tc_tiling_on_sc=False),
        scratch_types=(
            [pltpu.VMEM((kc, CC), jnp.int32)]
            + [pltpu.VMEM((CC, D), dt)] * 16
            + [pltpu.SemaphoreType.DMA, pltpu.SemaphoreType.DMA]
        ),
    )
    def k(table_hbm, idx_hbm, out_hbm, idx_v, *rest):
        bufs = rest[:16]
        gsem, wsem = rest[16], rest[17]
        cid = lax.axis_index("c")
        sid = lax.axis_index("s")
        wid = cid * 16 + sid
        pltpu.sync_copy(idx_hbm.at[wid], idx_v)
        base = wid * (kc * CC)
        half = (bufs[:8], bufs[8:])
        for b in range(8):
            pltpu.async_copy(table_hbm.at[idx_v.at[b]], bufs[b], gsem)

        def body(g, _):
            for par in range(2):
                @pl.when(lax.rem(g, 2) == par)
                def _():
                    cur = half[par]
                    nxt = half[1 - par]
                    for b in range(8):
                        pltpu.make_async_copy(
                            table_hbm.at[idx_v.at[0]], cur[b], gsem).wait()
                    @pl.when(g + 1 < ngrp)
                    def _():
                        @pl.when(g >= 1)
                        def _():
                            for b in range(8):
                                pltpu.make_async_copy(
                                    nxt[b], out_hbm.at[pl.ds(base, CC)],
                                    wsem).wait()
                        for b in range(8):
                            pltpu.async_copy(
                                table_hbm.at[idx_v.at[(g + 1) * 8 + b]],
                                nxt[b], gsem)
                    for b in range(8):
                        pltpu.async_copy(
                            cur[b],
                            out_hbm.at[pl.ds(base + (g * 8 + b) * CC, CC)],
                            wsem)
            return 0

        lax.fori_loop(0, ngrp, body, 0)
        for b in range(16):
            pltpu.make_async_copy(
                bufs[0], out_hbm.at[pl.ds(base, CC)], wsem).wait()

    return k(table, idx3)


def _sc_scatter_add16(values, idx3, zeros, K, S):
    PW = K * C
    TR = S // 16
    mesh = plsc.VectorSubcoreMesh(core_axis_name="c", subcore_axis_name="s")

    @functools.partial(
        pl.kernel,
        mesh=mesh,
        out_type=jax.ShapeDtypeStruct((2, S, 16), jnp.float32),
        compiler_params=pltpu.CompilerParams(use_tc_tiling_on_sc=False),
        scratch_types=[
            pltpu.VMEM((K, C), jnp.int32),
            pltpu.VMEM((PW, 16), jnp.float32),
            pltpu.VMEM((TR, 16), jnp.float32),
            pltpu.VMEM_SHARED((S, 16), jnp.float32),
        ],
    )
    def k(val_hbm, idx_hbm, z_hbm, out_hbm, idx_v, val_v, row_v, acc_sh):
        cid = lax.axis_index("c")
        sid = lax.axis_index("s")
        wid = cid * 16 + sid
        pltpu.sync_copy(z_hbm.at[pl.ds(sid * TR, TR)], row_v)
        pltpu.sync_copy(row_v, acc_sh.at[pl.ds(sid * TR, TR)])
        pltpu.sync_copy(idx_hbm.at[wid], idx_v)
        pltpu.sync_copy(val_hbm.at[pl.ds(wid * PW, PW)], val_v)
        plsc.subcore_barrier()

        def body(j, _):
            pltpu.sync_copy(val_v.at[pl.ds(j * C, C)],
                            acc_sh.at[idx_v.at[j]], add=True)
            return 0

        lax.fori_loop(0, K, body, 0)
        plsc.subcore_barrier()
        pltpu.sync_copy(acc_sh.at[pl.ds(sid * TR, TR)], row_v)
        pltpu.sync_copy(row_v, out_hbm.at[cid, pl.ds(sid * TR, TR)])

    return k(values, idx3, zeros)


def _msg_body(hs_ref, ea_ref, wp_ref, out_ref, *, din, dp):
    B = out_ref.shape[0]
    lanes = 8 * dp
    ea = ea_ref[...]
    w = jnp.dot(ea, wp_ref[...])
    w = jnp.maximum(w, 0.0).astype(jnp.bfloat16)
    hs = hs_ref[...].astype(jnp.bfloat16)
    if din < dp:
        hs = jnp.concatenate(
            [hs, jnp.zeros((B, dp - din), jnp.bfloat16)], axis=1)
    if dp == 128:
        t = jnp.concatenate(
            [hs * w[:, o * dp:(o + 1) * dp] for o in range(8)], axis=1)
    else:
        t = jnp.concatenate([hs] * 8, axis=1) * w
    ci = jax.lax.broadcasted_iota(jnp.int32, (lanes, 16), 0)
    oi = jax.lax.broadcasted_iota(jnp.int32, (lanes, 16), 1)
    sel = jnp.where((ci // dp) == oi, 1.0, 0.0).astype(jnp.bfloat16)
    msg = jnp.dot(t, sel, preferred_element_type=jnp.float32)
    onescol = (jax.lax.broadcasted_iota(jnp.int32, (B, 16), 1) == 8)
    out_ref[...] = msg + onescol.astype(jnp.float32)


def _tc_messages(hsrc, ea8, wp, din, dp):
    M = hsrc.shape[0]
    B = 4096 if dp == 128 else 8192
    grid = M // B
    return pl.pallas_call(
        functools.partial(_msg_body, din=din, dp=dp),
        grid=(grid,),
        compiler_params=pltpu.CompilerParams(
            vmem_limit_bytes=100 * 1024 * 1024),
        in_specs=[
            pl.BlockSpec((B, din), lambda i: (i, 0)),
            pl.BlockSpec((B, 8), lambda i: (i, 0)),
            pl.BlockSpec((8, 8 * dp), lambda i: (0, 0)),
        ],
        out_specs=pl.BlockSpec((B, 16), lambda i: (i, 0)),
        out_shape=jax.ShapeDtypeStruct((M, 16), jnp.float32),
    )(hsrc, ea8, wp)


def _combine_body(pa_ref, pb_ref, h_ref, root_ref, bias_ref, gamma_ref,
                  beta_ref, out_ref, *, din):
    s = pa_ref[0] + pa_ref[1] + pb_ref[0] + pb_ref[1]
    aggr = s[0:N, 0:8]
    deg = s[0:N, 8:9]
    denom = jnp.maximum(deg, 1.0)
    hp = h_ref[0:N, 0:din]
    out = aggr / denom + jnp.dot(hp, root_ref[...], precision=HIGH)
    out = out + bias_ref[...]
    mu = jnp.mean(out, axis=0, keepdims=True)
    var = jnp.mean((out - mu) ** 2, axis=0, keepdims=True)
    outn = (out - mu) * jax.lax.rsqrt(var + EPS)
    outn = outn * gamma_ref[...] + beta_ref[...]
    h = jnp.maximum(outn, 0.0)
    onescol = (jax.lax.broadcasted_iota(jnp.int32, (N, 8), 1) == 0)
    h16 = jnp.concatenate([h, onescol.astype(jnp.float32)], axis=1)
    pad = jnp.zeros((NPAD - N, 16), jnp.float32)
    out_ref[...] = jnp.concatenate([h16, pad], axis=0)


def _tc_combine(pa, pb, h_prev, rootp, bias, gamma, beta, din, hrows):
    return pl.pallas_call(
        functools.partial(_combine_body, din=din),
        compiler_params=pltpu.CompilerParams(
            vmem_limit_bytes=100 * 1024 * 1024),
        in_specs=[
            pl.BlockSpec((2, SN, 16), lambda: (0, 0, 0)),
            pl.BlockSpec((2, SN, 16), lambda: (0, 0, 0)),
            pl.BlockSpec((hrows, din), lambda: (0, 0)),
            pl.BlockSpec((din, HID), lambda: (0, 0)),
            pl.BlockSpec((1, HID), lambda: (0, 0)),
            pl.BlockSpec((1, HID), lambda: (0, 0)),
            pl.BlockSpec((1, HID), lambda: (0, 0)),
        ],
        out_specs=pl.BlockSpec((NPAD, 16), lambda: (0, 0)),
        out_shape=jax.ShapeDtypeStruct((NPAD, 16), jnp.float32),
    )(pa, pb, h_prev, rootp, bias, gamma, beta)


def _head_body(pp_ref, w_ref, b_ref, out_ref):
    s = pp_ref[0] + pp_ref[1]
    pooled = s[0:G, 0:8] / jnp.maximum(s[0:G, 8:9], 1.0)
    out_ref[...] = jnp.dot(pooled, w_ref[...], precision=HIGH) + b_ref[...]


def _tc_head(pp, headw, headb, n_tasks):
    return pl.pallas_call(
        _head_body,
        in_specs=[
            pl.BlockSpec((2, SG, 16), lambda: (0, 0, 0)),
            pl.BlockSpec((HID, n_tasks), lambda: (0, 0)),
            pl.BlockSpec((1, n_tasks), lambda: (0, 0)),
        ],
        out_specs=pl.BlockSpec((G, n_tasks), lambda: (0, 0)),
        out_shape=jax.ShapeDtypeStruct((G, n_tasks), jnp.float32),
    )(pp, headw, headb)


def _perm_edge_weights(we, be, in_ch, dp):
    we3 = we.reshape(EDGE_DIM, in_ch, HID)
    wp = jnp.transpose(we3, (0, 2, 1))
    wp = jnp.pad(wp, ((0, 0), (0, 0), (0, dp - in_ch)))
    bp = jnp.pad(be.reshape(in_ch, HID).T, ((0, 0), (0, dp - in_ch)))
    return jnp.concatenate([
        wp.reshape(EDGE_DIM, 8 * dp),
        bp.reshape(1, 8 * dp),
        jnp.zeros((3, 8 * dp), jnp.float32),
    ], axis=0)


def kernel(x, edge_index, edge_attr, batch, params):
    src = edge_index[0]
    dst = edge_index[1]
    EH = EPAD // 2
    src_f = jnp.pad(src, (0, EPAD - E))
    dst_f = jnp.pad(dst, (0, EPAD - E), constant_values=DEAD_N)
    src_h = [src_f[p * EH:(p + 1) * EH].reshape(NW, EH // NW // CC, CC)
             for p in range(2)]
    dst_h = [dst_f[p * EH:(p + 1) * EH].reshape(NW, EH // NW // C, C)
             for p in range(2)]
    ea8 = jnp.concatenate([
        edge_attr,
        jnp.ones((E, 1), jnp.float32),
        jnp.zeros((E, 3), jnp.float32),
    ], axis=1)
    ea8 = jnp.pad(ea8, ((0, EPAD - E), (0, 0)))
    ea8_h = [ea8[p * EH:(p + 1) * EH] for p in range(2)]
    batch_p = jnp.pad(batch, (0, NPAD - N),
                      constant_values=DEAD_G).reshape(NW, KN, C)
    z_n = jnp.zeros((SN, 16), jnp.float32)
    z_g = jnp.zeros((SG, 16), jnp.float32)

    h = x
    din = IN_DIM
    hrows = N
    for i in range(NUM_LAYERS):
        dp = 128 if din == IN_DIM else 16
        dgather = din if din == IN_DIM else 16
        wp = _perm_edge_weights(params["We"][i], params["be"][i], din, dp)
        rootp = params["root"][i]
        if din != IN_DIM:
            rootp = jnp.pad(rootp, ((0, 16 - HID), (0, 0)))
        hsrc_a = _sc_gather(h, src_h[0], dgather)
        hsrc_b = _sc_gather(h, src_h[1], dgather)
        msg_a = _tc_messages(hsrc_a, ea8_h[0], wp, dgather, dp)
        msg_b = _tc_messages(hsrc_b, ea8_h[1], wp, dgather, dp)
        part_a = _sc_scatter_add16(msg_a, dst_h[0], z_n, EH // NW // C, SN)
        part_b = _sc_scatter_add16(msg_b, dst_h[1], z_n, EH // NW // C, SN)
        h = _tc_combine(
            part_a, part_b, h, rootp,
            params["bias"][i].reshape(1, HID),
            params["gamma"][i].reshape(1, HID),
            params["beta"][i].reshape(1, HID),
            din if din == IN_DIM else 16, hrows)
        din = HID
        hrows = NPAD
    pp = _sc_scatter_add16(h, batch_p, z_g, KN, SG)
    n_tasks = params["headW"].shape[1]
    return _tc_head(pp, params["headW"], params["headb"].reshape(1, n_tasks),
                    n_tasks)
